# baseline (speedup 1.0000x reference)
import functools

import jax
import jax.numpy as jnp
from jax import lax
from jax.experimental import pallas as pl
from jax.experimental.pallas import tpu as pltpu
from jax.experimental.pallas import tpu_sc as plsc

_INFO = plsc.get_sparse_core_info()
NC = _INFO.num_cores
NS = _INFO.num_subcores
L = _INFO.num_lanes
NW = NC * NS

N_DST = 10000
D = 128
BATCH = 96
SHARE0 = 640
SHARE = 624
assert SHARE0 + (NS - 1) * SHARE == N_DST


def _i32(x):
    return jnp.int32(x)


def _sc_partial_sums(src_emb, sidx, didx, ept, nbf, tail):
    mesh = plsc.VectorSubcoreMesh(core_axis_name="c", subcore_axis_name="s")

    assert nbf >= 3 and 0 < tail <= L and tail % 8 == 0

    @functools.partial(
        pl.kernel,
        mesh=mesh,
        out_type=jax.ShapeDtypeStruct((NC, N_DST, D), jnp.float32),
        scratch_types=[
            pltpu.VMEM((5, 2, BATCH), jnp.int32),
            pltpu.VMEM((4, BATCH, D), jnp.float32),
            pltpu.VMEM_SHARED((N_DST, D), jnp.float32),
            pltpu.SemaphoreType.DMA,
            pltpu.SemaphoreType.DMA,
            pltpu.SemaphoreType.DMA,
            pltpu.SemaphoreType.DMA,
            pltpu.SemaphoreType.DMA,
            pltpu.SemaphoreType.DMA,
            pltpu.SemaphoreType.DMA,
            pltpu.SemaphoreType.DMA,
            pltpu.SemaphoreType.DMA,
            pltpu.SemaphoreType.DMA,
            pltpu.SemaphoreType.DMA,
            pltpu.SemaphoreType.DMA,
            pltpu.SemaphoreType.DMA,
        ],
    )
    def body(src_hbm, sidx_hbm, didx_hbm, out_hbm, ibufs, rows_v, acc_sh,
             is0, is1, is2, is3, is4,
             gs0, gs1, gs2, gs3, ss0, ss1, ss2, ss3):
        cid = lax.axis_index("c")
        sid = lax.axis_index("s")
        wid = sid * NC + cid
        tile_base = wid * ept
        isems = (is0, is1, is2, is3, is4)
        gsems = (gs0, gs1, gs2, gs3)
        ssems = (ss0, ss1, ss2, ss3)

        def _ifetch(jb, k):
            off = tile_base + jb * BATCH
            return (
                pltpu.make_async_copy(
                    sidx_hbm.at[pl.ds(off, BATCH)],
                    ibufs.at[_i32(k), _i32(0)], isems[k]),
                pltpu.make_async_copy(
                    didx_hbm.at[pl.ds(off, BATCH)],
                    ibufs.at[_i32(k), _i32(1)], isems[k]),
            )

        def _istart(jb, k):
            a, b = _ifetch(jb, k)
            a.start()
            b.start()

        def _iwait(jb, k):
            a, b = _ifetch(jb, k)
            a.wait()
            b.wait()

        def _gath(k8, r4):
            return pltpu.make_async_copy(
                src_hbm.at[ibufs.at[_i32(k8), _i32(0)]],
                rows_v.at[_i32(r4)], gsems[r4])

        def _scat_start(k8, r4):
            pltpu.async_copy(
                rows_v.at[_i32(r4)],
                acc_sh.at[ibufs.at[_i32(k8), _i32(1)]],
                ssems[r4], add=True)

        def _scat_wait(k8, r4):
            pltpu.make_async_copy(
                rows_v.at[_i32(r4)],
                acc_sh.at[ibufs.at[_i32(k8), _i32(1)]],
                ssems[r4]).wait()

        _istart(_i32(0), 0)
        _istart(_i32(1), 1)
        _istart(_i32(2), 2)

        @pl.loop(_i32(0), _i32(BATCH))
        def _zrow(i):
            for c in range(D // L):
                rows_v[_i32(2), i, pl.ds(c * L, L)] = jnp.zeros(
                    (L,), jnp.float32)

        def _zd(zbase, n):
            out = [pltpu.make_async_copy(
                       rows_v.at[_i32(2)],
                       acc_sh.at[pl.ds(zbase + k * BATCH, BATCH)], ss3)
                   for k in range(n // BATCH)]
            zrem = n % BATCH
            if zrem:
                out.append(pltpu.make_async_copy(
                    rows_v.at[_i32(2)].at[pl.ds(0, zrem)],
                    acc_sh.at[pl.ds(zbase + (n // BATCH) * BATCH, zrem)],
                    ss3))
            return out

        def _zdescs0():
            return _zd(0, SHARE0)

        def _zdescs():
            return _zd(SHARE0 + (sid - 1) * SHARE, SHARE)

        @pl.when(sid == 0)
        def _zero0():
            for c in _zdescs0():
                c.start()

        @pl.when(sid > 0)
        def _zero():
            for c in _zdescs():
                c.start()

        _iwait(_i32(0), 0)
        _gath(0, 0).start()
        _iwait(_i32(1), 1)
        _gath(1, 1).start()

        @pl.when(sid == 0)
        def _zwait0():
            for c in _zdescs0():
                c.wait()

        @pl.when(sid > 0)
        def _zwait():
            for c in _zdescs():
                c.wait()

        plsc.subcore_barrier()


        @pl.loop(_i32(0), _i32(nbf), step=_i32(20))
        def _step(j):
            for b in range(20):
                jb = j + b

                def _one(jb=jb, b=b):
                    _gath(b % 5, b % 4).wait()
                    _scat_start(b % 5, b % 4)

                    if b < 2:
                        @pl.when(jb > 1)
                        def _drain():
                            _scat_wait((b - 2) % 5, (b - 2) % 4)
                    else:
                        _scat_wait((b - 2) % 5, (b - 2) % 4)

                    @pl.when(jb + 3 < nbf)
                    def _pref():
                        _istart(jb + 3, (b + 3) % 5)

                    @pl.when(jb + 2 < nbf)
                    def _next():
                        _iwait(jb + 2, (b + 2) % 5)
                        _gath((b + 2) % 5, (b + 2) % 4).start()

                if b == 0:
                    _one()
                else:
                    pl.when(jb < nbf)(_one)

        _scat_wait((nbf - 2) % 5, (nbf - 2) % 4)
        _scat_wait((nbf - 1) % 5, (nbf - 1) % 4)
        toff = tile_base + nbf * BATCH
        pltpu.sync_copy(sidx_hbm.at[pl.ds(toff, tail)],
                        ibufs.at[_i32(0), _i32(0)].at[pl.ds(0, tail)])
        pltpu.sync_copy(didx_hbm.at[pl.ds(toff, tail)],
                        ibufs.at[_i32(0), _i32(1)].at[pl.ds(0, tail)])
        vs = ibufs[_i32(0), _i32(0), pl.ds(0, L)]
        vd = ibufs[_i32(0), _i32(1), pl.ds(0, L)]
        pltpu.async_copy(src_hbm.at[vs],
                         rows_v.at[_i32(0)].at[pl.ds(0, tail)], gs0).wait()
        pltpu.sync_copy(rows_v.at[_i32(0)].at[pl.ds(0, tail)],
                        acc_sh.at[vd], add=True)
        plsc.subcore_barrier()

        @pl.when(sid == 0)
        def _pub0():
            pltpu.sync_copy(acc_sh.at[pl.ds(0, SHARE0)],
                            out_hbm.at[cid, pl.ds(0, SHARE0)])

        @pl.when(sid > 0)
        def _pub():
            pbase = SHARE0 + (sid - 1) * SHARE
            pltpu.sync_copy(acc_sh.at[pl.ds(pbase, SHARE)],
                            out_hbm.at[cid, pl.ds(pbase, SHARE)])

    return body(src_emb, sidx, didx)


def _merge_partials(partials):
    blk = 2000

    def body(p_ref, o_ref):
        o_ref[...] = jnp.sum(p_ref[...], axis=0)

    return pl.pallas_call(
        body,
        out_shape=jax.ShapeDtypeStruct((N_DST, D), jnp.float32),
        grid=(N_DST // blk,),
        in_specs=[pl.BlockSpec((NC, blk, D), lambda i: (i * 0, i, i * 0))],
        out_specs=pl.BlockSpec((blk, D), lambda i: (i, i * 0)),
    )(partials)


def kernel(src_emb, edge_index):
    e = edge_index.shape[1]
    assert e % NW == 0
    ept = e // NW
    nbf = ept // BATCH
    tail = ept - nbf * BATCH

    sidx = edge_index[0].astype(jnp.int32)
    didx = edge_index[1].astype(jnp.int32)
    partials = _sc_partial_sums(src_emb, sidx, didx, ept, nbf, tail)
    return _merge_partials(partials)


# speedup vs baseline: 1.0049x; 1.0049x over previous
import functools

import jax
import jax.numpy as jnp
from jax import lax
from jax.experimental import pallas as pl
from jax.experimental.pallas import tpu as pltpu
from jax.experimental.pallas import tpu_sc as plsc

_INFO = plsc.get_sparse_core_info()
NC = _INFO.num_cores
NS = _INFO.num_subcores
L = _INFO.num_lanes
NW = NC * NS

N_DST = 10000
D = 128
BATCH = 128
SHARE0 = 640
SHARE = 624
assert SHARE0 + (NS - 1) * SHARE == N_DST


def _i32(x):
    return jnp.int32(x)


def _sc_partial_sums(src_emb, sidx, didx, ept, nbf, tail):
    mesh = plsc.VectorSubcoreMesh(core_axis_name="c", subcore_axis_name="s")

    assert nbf >= 3 and 0 < tail <= L and tail % 8 == 0

    @functools.partial(
        pl.kernel,
        mesh=mesh,
        out_type=jax.ShapeDtypeStruct((NC, N_DST, D), jnp.float32),
        scratch_types=[
            pltpu.VMEM((4, 2, BATCH), jnp.int32),
            pltpu.VMEM((3, BATCH, D), jnp.float32),
            pltpu.VMEM_SHARED((N_DST, D), jnp.float32),
            pltpu.SemaphoreType.DMA,
            pltpu.SemaphoreType.DMA,
            pltpu.SemaphoreType.DMA,
            pltpu.SemaphoreType.DMA,
            pltpu.SemaphoreType.DMA,
            pltpu.SemaphoreType.DMA,
            pltpu.SemaphoreType.DMA,
            pltpu.SemaphoreType.DMA,
            pltpu.SemaphoreType.DMA,
            pltpu.SemaphoreType.DMA,
        ],
    )
    def body(src_hbm, sidx_hbm, didx_hbm, out_hbm, ibufs, rows_v, acc_sh,
             is0, is1, is2, is3, gs0, gs1, gs2, ss0, ss1, ss2):
        cid = lax.axis_index("c")
        sid = lax.axis_index("s")
        wid = sid * NC + cid
        tile_base = wid * ept
        isems = (is0, is1, is2, is3)
        gsems = (gs0, gs1, gs2)
        ssems = (ss0, ss1, ss2)

        def _ifetch(jb, k):
            off = tile_base + jb * BATCH
            return (
                pltpu.make_async_copy(
                    sidx_hbm.at[pl.ds(off, BATCH)],
                    ibufs.at[_i32(k), _i32(0)], isems[k]),
                pltpu.make_async_copy(
                    didx_hbm.at[pl.ds(off, BATCH)],
                    ibufs.at[_i32(k), _i32(1)], isems[k]),
            )

        def _istart(jb, k):
            a, b = _ifetch(jb, k)
            a.start()
            b.start()

        def _iwait(jb, k):
            a, b = _ifetch(jb, k)
            a.wait()
            b.wait()

        def _gath(k4, r3):
            return pltpu.make_async_copy(
                src_hbm.at[ibufs.at[_i32(k4), _i32(0)]],
                rows_v.at[_i32(r3)], gsems[r3])

        def _scat_start(k4, r3):
            pltpu.async_copy(
                rows_v.at[_i32(r3)],
                acc_sh.at[ibufs.at[_i32(k4), _i32(1)]],
                ssems[r3], add=True)

        def _scat_wait(k4, r3):
            pltpu.make_async_copy(
                rows_v.at[_i32(r3)],
                acc_sh.at[ibufs.at[_i32(k4), _i32(1)]],
                ssems[r3]).wait()

        _istart(_i32(0), 0)
        _istart(_i32(1), 1)
        _istart(_i32(2), 2)

        @pl.loop(_i32(0), _i32(BATCH))
        def _zrow(i):
            for c in range(D // L):
                rows_v[_i32(2), i, pl.ds(c * L, L)] = jnp.zeros(
                    (L,), jnp.float32)

        def _zdescs0():
            return [pltpu.make_async_copy(
                        rows_v.at[_i32(2)],
                        acc_sh.at[pl.ds(k * BATCH, BATCH)], ss2)
                    for k in range(SHARE0 // BATCH)]

        def _zdescs():
            zbase = SHARE0 + (sid - 1) * SHARE
            out = [pltpu.make_async_copy(
                       rows_v.at[_i32(2)],
                       acc_sh.at[pl.ds(zbase + k * BATCH, BATCH)], ss2)
                   for k in range(SHARE // BATCH)]
            zrem = SHARE % BATCH
            if zrem:
                out.append(pltpu.make_async_copy(
                    rows_v.at[_i32(2)].at[pl.ds(0, zrem)],
                    acc_sh.at[pl.ds(zbase + (SHARE // BATCH) * BATCH, zrem)],
                    ss2))
            return out

        @pl.when(sid == 0)
        def _zero0():
            for c in _zdescs0():
                c.start()

        @pl.when(sid > 0)
        def _zero():
            for c in _zdescs():
                c.start()

        _iwait(_i32(0), 0)
        _gath(0, 0).start()
        _iwait(_i32(1), 1)
        _gath(1, 1).start()

        @pl.when(sid == 0)
        def _zwait0():
            for c in _zdescs0():
                c.wait()

        @pl.when(sid > 0)
        def _zwait():
            for c in _zdescs():
                c.wait()

        plsc.subcore_barrier()


        @pl.loop(_i32(0), _i32(nbf), step=_i32(12))
        def _step(j):
            for b in range(12):
                jb = j + b

                def _one(jb=jb, b=b):
                    _gath(b % 4, b % 3).wait()
                    _scat_start(b % 4, b % 3)

                    if b == 0:
                        @pl.when(jb > 0)
                        def _drain():
                            _scat_wait((b - 1) % 4, (b - 1) % 3)
                    else:
                        _scat_wait((b - 1) % 4, (b - 1) % 3)

                    @pl.when(jb + 3 < nbf)
                    def _pref():
                        _istart(jb + 3, (b + 3) % 4)

                    @pl.when(jb + 2 < nbf)
                    def _next():
                        _iwait(jb + 2, (b + 2) % 4)
                        _gath((b + 2) % 4, (b + 2) % 3).start()

                if b == 0:
                    _one()
                else:
                    pl.when(jb < nbf)(_one)

        _scat_wait((nbf - 1) % 4, (nbf - 1) % 3)
        toff = tile_base + nbf * BATCH
        pltpu.sync_copy(sidx_hbm.at[pl.ds(toff, tail)],
                        ibufs.at[_i32(0), _i32(0)].at[pl.ds(0, tail)])
        pltpu.sync_copy(didx_hbm.at[pl.ds(toff, tail)],
                        ibufs.at[_i32(0), _i32(1)].at[pl.ds(0, tail)])
        vs = ibufs[_i32(0), _i32(0), pl.ds(0, L)]
        vd = ibufs[_i32(0), _i32(1), pl.ds(0, L)]
        pltpu.async_copy(src_hbm.at[vs],
                         rows_v.at[_i32(0)].at[pl.ds(0, tail)], gs0).wait()
        pltpu.sync_copy(rows_v.at[_i32(0)].at[pl.ds(0, tail)],
                        acc_sh.at[vd], add=True)
        plsc.subcore_barrier()

        @pl.when(sid == 0)
        def _pub0():
            pltpu.sync_copy(acc_sh.at[pl.ds(0, SHARE0)],
                            out_hbm.at[cid, pl.ds(0, SHARE0)])

        @pl.when(sid > 0)
        def _pub():
            pbase = SHARE0 + (sid - 1) * SHARE
            pltpu.sync_copy(acc_sh.at[pl.ds(pbase, SHARE)],
                            out_hbm.at[cid, pl.ds(pbase, SHARE)])

    return body(src_emb, sidx, didx)


def _merge_partials(partials):
    blk = 2000

    def body(p_ref, o_ref):
        o_ref[...] = jnp.sum(p_ref[...], axis=0)

    return pl.pallas_call(
        body,
        out_shape=jax.ShapeDtypeStruct((N_DST, D), jnp.float32),
        grid=(N_DST // blk,),
        in_specs=[pl.BlockSpec((NC, blk, D), lambda i: (i * 0, i, i * 0))],
        out_specs=pl.BlockSpec((blk, D), lambda i: (i, i * 0)),
    )(partials)


def kernel(src_emb, edge_index):
    e = edge_index.shape[1]
    assert e % NW == 0
    ept = e // NW
    nbf = ept // BATCH
    tail = ept - nbf * BATCH

    sidx = edge_index[0].astype(jnp.int32)
    didx = edge_index[1].astype(jnp.int32)
    partials = _sc_partial_sums(src_emb, sidx, didx, ept, nbf, tail)
    return _merge_partials(partials)
